# baseline (speedup 1.0000x reference)
import functools

import jax
import jax.numpy as jnp
import numpy as np
from jax import lax
from jax.experimental import pallas as pl
from jax.experimental.pallas import tpu as pltpu
from jax.experimental.pallas import tpu_sc as plsc

N = 10000
E = 320000
D = 128
NR2 = 32
NEG = 33

NBKT = 128
SZ = 80
NPAD = NBKT * SZ
NTILES = 32
BPT = NBKT // NTILES
NPW = NPAD // NTILES
EB = 128

_mesh = plsc.VectorSubcoreMesh(core_axis_name="c", subcore_axis_name="s")


@functools.partial(
    pl.kernel,
    mesh=_mesh,
    out_type=(
        jax.ShapeDtypeStruct((NPAD, D), jnp.float32),
        jax.ShapeDtypeStruct((NPAD, D), jnp.float32),
        jax.ShapeDtypeStruct((NPAD, D), jnp.float32),
        jax.ShapeDtypeStruct((NPAD, D), jnp.float32),
        jax.ShapeDtypeStruct((NPAD, 16), jnp.float32),
    ),
    scratch_types=(
        pltpu.VMEM((SZ, D), jnp.float32),
        pltpu.VMEM((SZ, D), jnp.float32),
        pltpu.VMEM((SZ, D), jnp.float32),
        pltpu.VMEM((SZ, D), jnp.float32),
        pltpu.VMEM((SZ, 16), jnp.float32),
        pltpu.VMEM((NR2, D), jnp.float32),
        pltpu.VMEM((NPW + 16, ), jnp.int32),
        pltpu.VMEM((NPW + 16, ), jnp.int32),
        pltpu.VMEM((EB + 16,), jnp.uint32),
        pltpu.VMEM((EB + 16,), jnp.uint32),
        pltpu.VMEM((EB + 16,), jnp.int32),
        pltpu.VMEM((EB + 16,), jnp.int32),
        pltpu.VMEM((EB,), jnp.int32),
        pltpu.VMEM((EB,), jnp.int32),
        pltpu.VMEM((EB, D), jnp.float32),
        pltpu.VMEM((EB, D), jnp.float32),
        pltpu.SemaphoreType.DMA,
        pltpu.SemaphoreType.DMA,
        pltpu.SemaphoreType.DMA,
        pltpu.SemaphoreType.DMA,
    ),
)
def _edge_sc(x_hbm, rel_hbm, keys_hbm, srcs_hbm, attr_hbm, noffs_hbm,
             sum_hbm, sq_hbm, mx_hbm, mn_hbm, deg_hbm,
             acc_s, acc_q, acc_mx, acc_mn, acc_dg, rel_v, noffa_v, noffb_v,
             key0, key1, att0, att1, idx0, idx1, rows0, rows1,
             semk0, semk1, semg0, semg1):
    keyb = (key0, key1)
    attb = (att0, att1)
    idxb = (idx0, idx1)
    rowsb = (rows0, rows1)
    semk = (semk0, semk1)
    semg = (semg0, semg1)
    wid = lax.axis_index("s") * 2 + lax.axis_index("c")
    wbase = pl.multiple_of(wid * NPW, 8)
    pltpu.sync_copy(rel_hbm, rel_v)
    pltpu.sync_copy(noffs_hbm.at[pl.ds(wbase, NPW + 16)], noffa_v)
    pltpu.sync_copy(noffs_hbm.at[pl.ds(pl.multiple_of(NPAD + 16 + wbase, 8),
                                       NPW + 16)], noffb_v)
    zero = jnp.zeros((16,), jnp.float32)
    ninf = jnp.full((16,), -jnp.inf, jnp.float32)
    pinf = jnp.full((16,), jnp.inf, jnp.float32)

    def fire_ks(k, buf, ast):
        sta = pl.multiple_of(ast + k * EB, 8)
        pltpu.async_copy(keys_hbm.at[pl.ds(sta, EB)],
                         keyb[buf].at[pl.ds(0, EB)], semk[buf])
        pltpu.async_copy(attr_hbm.at[pl.ds(sta, EB)],
                         attb[buf].at[pl.ds(0, EB)], semk[buf])
        pltpu.async_copy(srcs_hbm.at[pl.ds(sta, EB)], idxb[buf], semk[buf])

    def wait_ks(k, buf, ast):
        sta = pl.multiple_of(ast + k * EB, 8)
        pltpu.make_async_copy(keys_hbm.at[pl.ds(sta, EB)],
                              keyb[buf].at[pl.ds(0, EB)], semk[buf]).wait()
        pltpu.make_async_copy(attr_hbm.at[pl.ds(sta, EB)],
                              attb[buf].at[pl.ds(0, EB)], semk[buf]).wait()
        pltpu.make_async_copy(srcs_hbm.at[pl.ds(sta, EB)], idxb[buf],
                              semk[buf]).wait()

    def fire_g(buf):
        pltpu.async_copy(x_hbm.at[idxb[buf]], rowsb[buf], semg[buf])

    def wait_g(buf):
        pltpu.make_async_copy(x_hbm.at[idxb[buf]], rowsb[buf],
                              semg[buf]).wait()

    @pl.loop(wid * BPT, wid * BPT + BPT)
    def _bucket(b):
        base = pl.multiple_of(b * SZ, 8)
        lb = (b - wid * BPT) * SZ

        @plsc.parallel_loop(0, SZ)
        def _zrow(r):
            for j in range(8):
                sl = pl.ds(16 * j, 16)
                acc_s[r, sl] = zero
                acc_q[r, sl] = zero
                acc_mx[r, sl] = ninf
                acc_mn[r, sl] = pinf
            acc_dg[r, :] = zero

        def run_half(nof):
            start = nof[pl.ds(lb, 16)][0]
            stop = nof[pl.ds(lb + SZ, 16)][0]
            ast = (start // 8) * 8
            nb = (stop - ast + EB - 1) // EB

            @pl.when(nb > 0)
            def _prologue():
                fire_ks(0, 0, ast)
                wait_ks(0, 0, ast)
                fire_g(0)

                @pl.when(nb > 1)
                def _():
                    fire_ks(1, 1, ast)

            nbp = (nb + 1) // 2

            @pl.loop(0, nbp)
            def _pair(i):
                for half in range(2):
                    kk = i * 2 + half
                    buf = half

                    @pl.when(kk < nb)
                    def _do():
                        wait_g(buf)

                        @pl.when(kk + 1 < nb)
                        def _():
                            wait_ks(kk + 1, 1 - buf, ast)
                            fire_g(1 - buf)

                        st = ast + kk * EB
                        lo = jnp.maximum(start - st, 0)
                        hi = jnp.minimum(stop - st, EB)

                        @pl.when(hi > lo)
                        def _compute():
                            dlo = (keyb[buf][pl.ds(lo, 16)][0] >> 18).astype(
                                jnp.int32)
                            dhi = (keyb[buf][pl.ds(hi - 1, 16)][0]
                                   >> 18).astype(jnp.int32)

                            @pl.loop(dlo, dhi + 1)
                            def _node(d):
                                dl = d - base
                                nv = nof[pl.ds(d - wbase, 16)]
                                elo = jnp.maximum(nv[0] - st, 0)
                                ehi = jnp.minimum(nv[1] - st, EB)
                                init = (tuple(zero for _ in range(8)),
                                        tuple(zero for _ in range(8)),
                                        tuple(ninf for _ in range(8)),
                                        tuple(pinf for _ in range(8)))

                                @plsc.parallel_loop(elo, ehi, carry=init)
                                def _edge(e, c):
                                    cs, cq, cmx, cmn = c
                                    a = attb[buf][pl.ds(e, 16)][0]
                                    ns, nq, nmx, nmn = [], [], [], []
                                    for j in range(8):
                                        sl = pl.ds(16 * j, 16)
                                        m = rowsb[buf][e, sl] * rel_v[a, sl]
                                        ns.append(cs[j] + m)
                                        nq.append(cq[j] + m * m)
                                        nmx.append(jnp.maximum(cmx[j], m))
                                        nmn.append(jnp.minimum(cmn[j], m))
                                    return (tuple(ns), tuple(nq), tuple(nmx),
                                            tuple(nmn))

                                fs, fq, fmx, fmn = _edge
                                for j in range(8):
                                    sl = pl.ds(16 * j, 16)
                                    acc_s[dl, sl] = acc_s[dl, sl] + fs[j]
                                    acc_q[dl, sl] = acc_q[dl, sl] + fq[j]
                                    acc_mx[dl, sl] = jnp.maximum(
                                        acc_mx[dl, sl], fmx[j])
                                    acc_mn[dl, sl] = jnp.minimum(
                                        acc_mn[dl, sl], fmn[j])
                                cntf = (ehi - elo).astype(jnp.float32)
                                acc_dg[dl, :] = acc_dg[dl, :] + cntf

                        @pl.when(kk + 2 < nb)
                        def _refill():
                            fire_ks(kk + 2, buf, ast)

        run_half(noffa_v)
        run_half(noffb_v)

        pltpu.sync_copy(acc_s.at[pl.ds(0, SZ)], sum_hbm.at[pl.ds(base, SZ)])
        pltpu.sync_copy(acc_q.at[pl.ds(0, SZ)], sq_hbm.at[pl.ds(base, SZ)])
        pltpu.sync_copy(acc_mx.at[pl.ds(0, SZ)], mx_hbm.at[pl.ds(base, SZ)])
        pltpu.sync_copy(acc_mn.at[pl.ds(0, SZ)], mn_hbm.at[pl.ds(base, SZ)])
        pltpu.sync_copy(acc_dg.at[pl.ds(0, SZ)], deg_hbm.at[pl.ds(base, SZ)])


def _node_tc(x_ref, s_ref, q_ref, mx_ref, mn_ref, aux_ref, qry_ref, w_ref,
             b_ref, o_ref):
    aux = aux_ref[...]
    invd = aux[:, 0:1]
    s1 = aux[:, 1:2]
    s2 = aux[:, 2:3]
    cnt = aux[:, 3:4]
    qv = qry_ref[0:1, :]
    bnd = cnt * qv
    mean = (s_ref[...] + bnd) * invd
    sqm = (q_ref[...] + bnd * bnd) * invd
    mxv = jnp.maximum(mx_ref[...], bnd)
    mnv = jnp.minimum(mn_ref[...], bnd)
    std = jnp.sqrt(jnp.clip(sqm - mean * mean, 1e-6, None))
    acc = jnp.dot(x_ref[...], w_ref[0:D, :], preferred_element_type=jnp.float32)
    k = 1
    for stat in (mean, mxv, mnv, std):
        acc = acc + jnp.dot(stat, w_ref[k * D:(k + 1) * D, :],
                            preferred_element_type=jnp.float32)
        k += 1
        acc = acc + jnp.dot(stat * s1, w_ref[k * D:(k + 1) * D, :],
                            preferred_element_type=jnp.float32)
        k += 1
        acc = acc + jnp.dot(stat * s2, w_ref[k * D:(k + 1) * D, :],
                            preferred_element_type=jnp.float32)
        k += 1
    o_ref[...] = jnp.maximum(acc + b_ref[0:1, :], 0.0)


def _node_call(x, s, q, mx, mn, aux, q8, wre, b8):
    blk = 64
    grid = NPAD // blk

    def bs():
        return pl.BlockSpec((blk, D), lambda i: (i, 0))

    return pl.pallas_call(
        _node_tc,
        grid=(grid,),
        in_specs=[bs(), bs(), bs(), bs(), bs(), bs(),
                  pl.BlockSpec((8, D), lambda i: (0, 0)),
                  pl.BlockSpec((13 * D, D), lambda i: (0, 0)),
                  pl.BlockSpec((8, D), lambda i: (0, 0))],
        out_specs=bs(),
        out_shape=jax.ShapeDtypeStruct((NPAD, D), jnp.float32),
    )(x, s, q, mx, mn, aux, q8, wre, b8)


def _rel_tc(q_ref, w_ref, b_ref, o_ref):
    o_ref[...] = jnp.dot(q_ref[...], w_ref[...],
                         preferred_element_type=jnp.float32) + b_ref[...]


def _mlp_tc(t_ref, q_ref, wa_ref, wb_ref, bl_ref, w1_ref, b1_ref, w2_ref,
            b2_ref, o_ref):
    qrow = q_ref[0:1, :]
    h = (jnp.dot(t_ref[...], wa_ref[...], preferred_element_type=jnp.float32)
         + jnp.dot(qrow, wb_ref[...], preferred_element_type=jnp.float32)
         + bl_ref[0:1, :])
    h = jnp.maximum(h, 0.0)
    h = jnp.maximum(jnp.dot(h, w1_ref[...], preferred_element_type=jnp.float32)
                    + b1_ref[0:1, :], 0.0)
    o_ref[...] = jnp.dot(h, w2_ref[...],
                         preferred_element_type=jnp.float32) + b2_ref[0:1, :]


_PERM = np.concatenate(
    [np.arange(D)]
    + [D + 12 * np.arange(D) + 3 * c + j for c in range(4) for j in range(3)])


def kernel(h_index, r_index, t_index, hidden_states, rel_hidden_states,
           edge_index, edge_attr, score_text_embs, all_index, Wr0, br0, Wl0,
           bl0, Wr1, br1, Wl1, bl1, Wlin, blin, Wm1, bm1, Wm2, bm2):
    f32 = jnp.float32
    query = rel_hidden_states[r_index[0, 0]]
    q8 = jnp.broadcast_to(query[None, :], (8, D))

    wcat = jnp.concatenate([Wr0, Wr1], axis=1)
    bcat = jnp.broadcast_to(jnp.concatenate([br0, br1])[None, :], (8, 2 * NR2 * D))
    rel_out = pl.pallas_call(
        _rel_tc,
        out_shape=jax.ShapeDtypeStruct((8, 2 * NR2 * D), f32),
    )(q8, wcat, bcat)
    rel0 = rel_out[0, :NR2 * D].reshape(NR2, D)
    rel1 = rel_out[0, NR2 * D:].reshape(NR2, D)

    src = edge_index[0]
    dst = edge_index[1]
    EH = E // 2
    eidh = jnp.arange(EH, dtype=jnp.uint32)
    kA = (dst[:EH].astype(jnp.uint32) << 18) | eidh
    kB = (dst[EH:].astype(jnp.uint32) << 18) | eidh
    skA = lax.sort(kA)
    skB = lax.sort(kB)
    mask = jnp.uint32(0x3FFFF)
    seid = jnp.concatenate([(skA & mask).astype(jnp.int32),
                            (skB & mask).astype(jnp.int32) + EH])
    ssrc = src[seid]
    sattr = edge_attr[seid]
    nbounds = jnp.arange(NPAD + 1, dtype=jnp.uint32) << 18
    NOF = NPAD + 16
    noffsA = jnp.searchsorted(skA, nbounds).astype(jnp.int32)
    noffsB = jnp.searchsorted(skB, nbounds).astype(jnp.int32) + EH
    noffs_p = jnp.concatenate([
        jnp.full((NOF,), EH, jnp.int32).at[:NPAD + 1].set(noffsA),
        jnp.full((NOF,), E, jnp.int32).at[:NPAD + 1].set(noffsB)])
    skp = jnp.pad(jnp.concatenate([skA, skB]), (0, 2 * EB))
    ssp = jnp.pad(ssrc, (0, 2 * EB))
    sap = jnp.pad(sattr, (0, 2 * EB))

    cnt = jnp.zeros((NPAD, 1), f32).at[h_index[0]].add(1.0)
    x0 = jnp.pad(hidden_states + score_text_embs, ((0, NPAD - N), (0, 0)))

    wre0 = Wl0[_PERM]
    wre1 = Wl1[_PERM]
    b0_8 = jnp.broadcast_to(bl0[None, :], (8, D))
    b1_8 = jnp.broadcast_to(bl1[None, :], (8, D))

    s0, q0, mx0, mn0, dg0 = _edge_sc(x0, rel0, skp, ssp, sap, noffs_p)

    degf = dg0[:, 0:1] + 1.0
    sm = jnp.mean(jnp.log(degf[:N]))
    scale = jnp.log(degf) / (sm + 1e-10)
    aux4 = jnp.concatenate(
        [1.0 / degf, scale, 1.0 / jnp.clip(scale, 0.01, None), cnt], axis=1)
    aux = jnp.pad(aux4, ((0, 0), (0, D - 4)))

    x1 = _node_call(x0, s0, q0, mx0, mn0, aux, q8, wre0, b0_8)

    s1_, q1_, mx1, mn1, _ = _edge_sc(x1, rel1, skp, ssp, sap, noffs_p)
    x2 = _node_call(x1, s1_, q1_, mx1, mn1, aux, q8, wre1, b1_8)

    tails = jnp.pad(x2[t_index[0]], ((0, 40 - NEG), (0, 0)))
    w2p = jnp.pad(Wm2, ((0, 0), (0, D - 1)))
    b2p = jnp.broadcast_to(jnp.pad(bm2, (0, D - 1))[None, :], (8, D))
    out = pl.pallas_call(
        _mlp_tc,
        out_shape=jax.ShapeDtypeStruct((40, D), f32),
    )(tails, q8, Wlin[:D], Wlin[D:], jnp.broadcast_to(blin[None, :], (8, D)),
      Wm1, jnp.broadcast_to(bm1[None, :], (8, 2 * D)), w2p, b2p)
    return out[:NEG, 0].reshape(1, NEG)


# speedup vs baseline: 1.5251x; 1.5251x over previous
import functools

import jax
import jax.numpy as jnp
import numpy as np
from jax import lax
from jax.experimental import pallas as pl
from jax.experimental.pallas import tpu as pltpu
from jax.experimental.pallas import tpu_sc as plsc

N = 10000
E = 320000
D = 128
NR2 = 32
NEG = 33

NBKT = 128
SZ = 80
NPAD = NBKT * SZ
NTILES = 32
BPT = NBKT // NTILES
NPW = NPAD // NTILES
EB = 128

_mesh = plsc.VectorSubcoreMesh(core_axis_name="c", subcore_axis_name="s")


@functools.partial(
    pl.kernel,
    mesh=_mesh,
    out_type=(
        jax.ShapeDtypeStruct((NPAD, D), jnp.float32),
        jax.ShapeDtypeStruct((NPAD, D), jnp.float32),
        jax.ShapeDtypeStruct((NPAD, D), jnp.float32),
        jax.ShapeDtypeStruct((NPAD, D), jnp.float32),
        jax.ShapeDtypeStruct((NPAD, 16), jnp.float32),
    ),
    scratch_types=(
        pltpu.VMEM((SZ, D), jnp.float32),
        pltpu.VMEM((SZ, D), jnp.float32),
        pltpu.VMEM((SZ, D), jnp.float32),
        pltpu.VMEM((SZ, D), jnp.float32),
        pltpu.VMEM((SZ, 16), jnp.float32),
        pltpu.VMEM((NR2, D), jnp.float32),
        pltpu.VMEM((NPW + 16, ), jnp.int32),
        pltpu.VMEM((EB + 16,), jnp.int32),
        pltpu.VMEM((EB + 16,), jnp.int32),
        pltpu.VMEM((EB + 16,), jnp.int32),
        pltpu.VMEM((EB + 16,), jnp.int32),
        pltpu.VMEM((EB,), jnp.int32),
        pltpu.VMEM((EB,), jnp.int32),
        pltpu.VMEM((EB, D), jnp.float32),
        pltpu.VMEM((EB, D), jnp.float32),
        pltpu.SemaphoreType.DMA,
        pltpu.SemaphoreType.DMA,
        pltpu.SemaphoreType.DMA,
        pltpu.SemaphoreType.DMA,
    ),
)
def _edge_sc(x_hbm, rel_hbm, keys_hbm, srcs_hbm, attr_hbm, noffs_hbm,
             sum_hbm, sq_hbm, mx_hbm, mn_hbm, deg_hbm,
             acc_s, acc_q, acc_mx, acc_mn, acc_dg, rel_v, noffs_v,
             key0, key1, att0, att1, idx0, idx1, rows0, rows1,
             semk0, semk1, semg0, semg1):
    keyb = (key0, key1)
    attb = (att0, att1)
    idxb = (idx0, idx1)
    rowsb = (rows0, rows1)
    semk = (semk0, semk1)
    semg = (semg0, semg1)
    wid = lax.axis_index("s") * 2 + lax.axis_index("c")
    wbase = pl.multiple_of(wid * NPW, 8)
    pltpu.sync_copy(rel_hbm, rel_v)
    pltpu.sync_copy(noffs_hbm.at[pl.ds(wbase, NPW + 16)], noffs_v)
    zero = jnp.zeros((16,), jnp.float32)
    ninf = jnp.full((16,), -jnp.inf, jnp.float32)
    pinf = jnp.full((16,), jnp.inf, jnp.float32)

    def fire_ks(k, buf, ast):
        sta = pl.multiple_of(ast + k * EB, 8)
        pltpu.async_copy(keys_hbm.at[pl.ds(sta, EB)],
                         keyb[buf].at[pl.ds(0, EB)], semk[buf])
        pltpu.async_copy(attr_hbm.at[pl.ds(sta, EB)],
                         attb[buf].at[pl.ds(0, EB)], semk[buf])
        pltpu.async_copy(srcs_hbm.at[pl.ds(sta, EB)], idxb[buf], semk[buf])

    def wait_ks(k, buf, ast):
        sta = pl.multiple_of(ast + k * EB, 8)
        pltpu.make_async_copy(keys_hbm.at[pl.ds(sta, EB)],
                              keyb[buf].at[pl.ds(0, EB)], semk[buf]).wait()
        pltpu.make_async_copy(attr_hbm.at[pl.ds(sta, EB)],
                              attb[buf].at[pl.ds(0, EB)], semk[buf]).wait()
        pltpu.make_async_copy(srcs_hbm.at[pl.ds(sta, EB)], idxb[buf],
                              semk[buf]).wait()

    def fire_g(buf):
        pltpu.async_copy(x_hbm.at[idxb[buf]], rowsb[buf], semg[buf])

    def wait_g(buf):
        pltpu.make_async_copy(x_hbm.at[idxb[buf]], rowsb[buf],
                              semg[buf]).wait()

    @pl.loop(wid * BPT, wid * BPT + BPT)
    def _bucket(b):
        base = pl.multiple_of(b * SZ, 8)
        lb = (b - wid * BPT) * SZ

        @plsc.parallel_loop(0, SZ)
        def _zrow(r):
            for j in range(8):
                sl = pl.ds(16 * j, 16)
                acc_s[r, sl] = zero
                acc_q[r, sl] = zero
                acc_mx[r, sl] = ninf
                acc_mn[r, sl] = pinf
            acc_dg[r, :] = zero

        def run_half(nof):
            start = nof[pl.ds(lb, 16)][0]
            stop = nof[pl.ds(lb + SZ, 16)][0]
            ast = (start // 8) * 8
            nb = (stop - ast + EB - 1) // EB

            @pl.when(nb > 0)
            def _prologue():
                fire_ks(0, 0, ast)
                wait_ks(0, 0, ast)
                fire_g(0)

                @pl.when(nb > 1)
                def _():
                    fire_ks(1, 1, ast)

            nbp = (nb + 1) // 2

            @pl.loop(0, nbp)
            def _pair(i):
                for half in range(2):
                    kk = i * 2 + half
                    buf = half

                    @pl.when(kk < nb)
                    def _do():
                        wait_g(buf)

                        @pl.when(kk + 1 < nb)
                        def _():
                            wait_ks(kk + 1, 1 - buf, ast)
                            fire_g(1 - buf)

                        st = ast + kk * EB
                        lo = jnp.maximum(start - st, 0)
                        hi = jnp.minimum(stop - st, EB)

                        @pl.when(hi > lo)
                        def _compute():
                            dlo = keyb[buf][pl.ds(lo, 16)][0]
                            dhi = keyb[buf][pl.ds(hi - 1, 16)][0]

                            @pl.loop(dlo, dhi + 1)
                            def _node(d):
                                dl = d - base
                                nv = nof[pl.ds(d - wbase, 16)]
                                elo = jnp.maximum(nv[0] - st, 0)
                                ehi = jnp.minimum(nv[1] - st, EB)
                                init = (tuple(zero for _ in range(8)),
                                        tuple(zero for _ in range(8)),
                                        tuple(ninf for _ in range(8)),
                                        tuple(pinf for _ in range(8)))

                                @plsc.parallel_loop(elo, ehi, carry=init)
                                def _edge(e, c):
                                    cs, cq, cmx, cmn = c
                                    a = attb[buf][pl.ds(e, 16)][0]
                                    ns, nq, nmx, nmn = [], [], [], []
                                    for j in range(8):
                                        sl = pl.ds(16 * j, 16)
                                        m = rowsb[buf][e, sl] * rel_v[a, sl]
                                        ns.append(cs[j] + m)
                                        nq.append(cq[j] + m * m)
                                        nmx.append(jnp.maximum(cmx[j], m))
                                        nmn.append(jnp.minimum(cmn[j], m))
                                    return (tuple(ns), tuple(nq), tuple(nmx),
                                            tuple(nmn))

                                fs, fq, fmx, fmn = _edge
                                for j in range(8):
                                    sl = pl.ds(16 * j, 16)
                                    acc_s[dl, sl] = acc_s[dl, sl] + fs[j]
                                    acc_q[dl, sl] = acc_q[dl, sl] + fq[j]
                                    acc_mx[dl, sl] = jnp.maximum(
                                        acc_mx[dl, sl], fmx[j])
                                    acc_mn[dl, sl] = jnp.minimum(
                                        acc_mn[dl, sl], fmn[j])
                                cntf = (ehi - elo).astype(jnp.float32)
                                acc_dg[dl, :] = acc_dg[dl, :] + cntf

                        @pl.when(kk + 2 < nb)
                        def _refill():
                            fire_ks(kk + 2, buf, ast)

        run_half(noffs_v)

        pltpu.sync_copy(acc_s.at[pl.ds(0, SZ)], sum_hbm.at[pl.ds(base, SZ)])
        pltpu.sync_copy(acc_q.at[pl.ds(0, SZ)], sq_hbm.at[pl.ds(base, SZ)])
        pltpu.sync_copy(acc_mx.at[pl.ds(0, SZ)], mx_hbm.at[pl.ds(base, SZ)])
        pltpu.sync_copy(acc_mn.at[pl.ds(0, SZ)], mn_hbm.at[pl.ds(base, SZ)])
        pltpu.sync_copy(acc_dg.at[pl.ds(0, SZ)], deg_hbm.at[pl.ds(base, SZ)])


def _node_tc(x_ref, s_ref, q_ref, mx_ref, mn_ref, aux_ref, qry_ref, w_ref,
             b_ref, o_ref):
    aux = aux_ref[...]
    invd = aux[:, 0:1]
    s1 = aux[:, 1:2]
    s2 = aux[:, 2:3]
    cnt = aux[:, 3:4]
    qv = qry_ref[0:1, :]
    bnd = cnt * qv
    mean = (s_ref[...] + bnd) * invd
    sqm = (q_ref[...] + bnd * bnd) * invd
    mxv = jnp.maximum(mx_ref[...], bnd)
    mnv = jnp.minimum(mn_ref[...], bnd)
    std = jnp.sqrt(jnp.clip(sqm - mean * mean, 1e-6, None))
    acc = jnp.dot(x_ref[...], w_ref[0:D, :], preferred_element_type=jnp.float32)
    k = 1
    for stat in (mean, mxv, mnv, std):
        acc = acc + jnp.dot(stat, w_ref[k * D:(k + 1) * D, :],
                            preferred_element_type=jnp.float32)
        k += 1
        acc = acc + jnp.dot(stat * s1, w_ref[k * D:(k + 1) * D, :],
                            preferred_element_type=jnp.float32)
        k += 1
        acc = acc + jnp.dot(stat * s2, w_ref[k * D:(k + 1) * D, :],
                            preferred_element_type=jnp.float32)
        k += 1
    o_ref[...] = jnp.maximum(acc + b_ref[0:1, :], 0.0)


def _node_call(x, s, q, mx, mn, aux, q8, wre, b8):
    blk = 64
    grid = NPAD // blk

    def bs():
        return pl.BlockSpec((blk, D), lambda i: (i, 0))

    return pl.pallas_call(
        _node_tc,
        grid=(grid,),
        in_specs=[bs(), bs(), bs(), bs(), bs(), bs(),
                  pl.BlockSpec((8, D), lambda i: (0, 0)),
                  pl.BlockSpec((13 * D, D), lambda i: (0, 0)),
                  pl.BlockSpec((8, D), lambda i: (0, 0))],
        out_specs=bs(),
        out_shape=jax.ShapeDtypeStruct((NPAD, D), jnp.float32),
    )(x, s, q, mx, mn, aux, q8, wre, b8)


def _rel_tc(q_ref, w_ref, b_ref, o_ref):
    o_ref[...] = jnp.dot(q_ref[...], w_ref[...],
                         preferred_element_type=jnp.float32) + b_ref[...]


def _mlp_tc(t_ref, q_ref, wa_ref, wb_ref, bl_ref, w1_ref, b1_ref, w2_ref,
            b2_ref, o_ref):
    qrow = q_ref[0:1, :]
    h = (jnp.dot(t_ref[...], wa_ref[...], preferred_element_type=jnp.float32)
         + jnp.dot(qrow, wb_ref[...], preferred_element_type=jnp.float32)
         + bl_ref[0:1, :])
    h = jnp.maximum(h, 0.0)
    h = jnp.maximum(jnp.dot(h, w1_ref[...], preferred_element_type=jnp.float32)
                    + b1_ref[0:1, :], 0.0)
    o_ref[...] = jnp.dot(h, w2_ref[...],
                         preferred_element_type=jnp.float32) + b2_ref[0:1, :]


_PERM = np.concatenate(
    [np.arange(D)]
    + [D + 12 * np.arange(D) + 3 * c + j for c in range(4) for j in range(3)])


def kernel(h_index, r_index, t_index, hidden_states, rel_hidden_states,
           edge_index, edge_attr, score_text_embs, all_index, Wr0, br0, Wl0,
           bl0, Wr1, br1, Wl1, bl1, Wlin, blin, Wm1, bm1, Wm2, bm2):
    f32 = jnp.float32
    query = rel_hidden_states[r_index[0, 0]]
    q8 = jnp.broadcast_to(query[None, :], (8, D))

    wcat = jnp.concatenate([Wr0, Wr1], axis=1)
    bcat = jnp.broadcast_to(jnp.concatenate([br0, br1])[None, :], (8, 2 * NR2 * D))
    rel_out = pl.pallas_call(
        _rel_tc,
        out_shape=jax.ShapeDtypeStruct((8, 2 * NR2 * D), f32),
    )(q8, wcat, bcat)
    rel0 = rel_out[0, :NR2 * D].reshape(NR2, D)
    rel1 = rel_out[0, NR2 * D:].reshape(NR2, D)

    src = edge_index[0]
    dst = edge_index[1]
    keys = (dst.astype(jnp.uint32) * jnp.uint32(E)
            + src.astype(jnp.uint32) * jnp.uint32(32)
            + edge_attr.astype(jnp.uint32))
    sk = lax.sort(keys)
    rem = sk % jnp.uint32(E)
    sdst = (sk // jnp.uint32(E)).astype(jnp.int32)
    ssrc = (rem // jnp.uint32(32)).astype(jnp.int32)
    sattr = (rem % jnp.uint32(32)).astype(jnp.int32)
    nbounds = jnp.arange(NPAD + 1, dtype=jnp.uint32) * jnp.uint32(E)
    noffs = jnp.searchsorted(sk, nbounds).astype(jnp.int32)
    noffs_p = jnp.full((NPAD + 16,), E, jnp.int32).at[:NPAD + 1].set(noffs)
    skp = jnp.pad(sdst, (0, 2 * EB))
    ssp = jnp.pad(ssrc, (0, 2 * EB))
    sap = jnp.pad(sattr, (0, 2 * EB))

    cnt = jnp.zeros((NPAD, 1), f32).at[h_index[0]].add(1.0)
    x0 = jnp.pad(hidden_states + score_text_embs, ((0, NPAD - N), (0, 0)))

    wre0 = Wl0[_PERM]
    wre1 = Wl1[_PERM]
    b0_8 = jnp.broadcast_to(bl0[None, :], (8, D))
    b1_8 = jnp.broadcast_to(bl1[None, :], (8, D))

    s0, q0, mx0, mn0, dg0 = _edge_sc(x0, rel0, skp, ssp, sap, noffs_p)

    degf = dg0[:, 0:1] + 1.0
    sm = jnp.mean(jnp.log(degf[:N]))
    scale = jnp.log(degf) / (sm + 1e-10)
    aux4 = jnp.concatenate(
        [1.0 / degf, scale, 1.0 / jnp.clip(scale, 0.01, None), cnt], axis=1)
    aux = jnp.pad(aux4, ((0, 0), (0, D - 4)))

    x1 = _node_call(x0, s0, q0, mx0, mn0, aux, q8, wre0, b0_8)

    s1_, q1_, mx1, mn1, _ = _edge_sc(x1, rel1, skp, ssp, sap, noffs_p)
    x2 = _node_call(x1, s1_, q1_, mx1, mn1, aux, q8, wre1, b1_8)

    tails = jnp.pad(x2[t_index[0]], ((0, 40 - NEG), (0, 0)))
    w2p = jnp.pad(Wm2, ((0, 0), (0, D - 1)))
    b2p = jnp.broadcast_to(jnp.pad(bm2, (0, D - 1))[None, :], (8, D))
    out = pl.pallas_call(
        _mlp_tc,
        out_shape=jax.ShapeDtypeStruct((40, D), f32),
    )(tails, q8, Wlin[:D], Wlin[D:], jnp.broadcast_to(blin[None, :], (8, D)),
      Wm1, jnp.broadcast_to(bm1[None, :], (8, 2 * D)), w2p, b2p)
    return out[:NEG, 0].reshape(1, NEG)


# speedup vs baseline: 1.5254x; 1.0002x over previous
import functools

import jax
import jax.numpy as jnp
import numpy as np
from jax import lax
from jax.experimental import pallas as pl
from jax.experimental.pallas import tpu as pltpu
from jax.experimental.pallas import tpu_sc as plsc

N = 10000
E = 320000
D = 128
NR2 = 32
NEG = 33

NBKT = 128
SZ = 80
NPAD = NBKT * SZ
NTILES = 32
BPT = NBKT // NTILES
NPW = NPAD // NTILES
EB = 128

_mesh = plsc.VectorSubcoreMesh(core_axis_name="c", subcore_axis_name="s")


@functools.partial(
    pl.kernel,
    mesh=_mesh,
    out_type=(
        jax.ShapeDtypeStruct((NPAD, D), jnp.float32),
        jax.ShapeDtypeStruct((NPAD, D), jnp.float32),
        jax.ShapeDtypeStruct((NPAD, D), jnp.float32),
        jax.ShapeDtypeStruct((NPAD, D), jnp.float32),
        jax.ShapeDtypeStruct((NPAD, 16), jnp.float32),
    ),
    scratch_types=(
        pltpu.VMEM((SZ, D), jnp.float32),
        pltpu.VMEM((SZ, D), jnp.float32),
        pltpu.VMEM((SZ, D), jnp.float32),
        pltpu.VMEM((SZ, D), jnp.float32),
        pltpu.VMEM((SZ, 16), jnp.float32),
        pltpu.VMEM((NR2, D), jnp.float32),
        pltpu.VMEM((NPW + 16, ), jnp.int32),
        pltpu.VMEM((EB + 16,), jnp.int32),
        pltpu.VMEM((EB + 16,), jnp.int32),
        pltpu.VMEM((EB + 16,), jnp.int32),
        pltpu.VMEM((EB + 16,), jnp.int32),
        pltpu.VMEM((EB,), jnp.int32),
        pltpu.VMEM((EB,), jnp.int32),
        pltpu.VMEM((EB, D), jnp.float32),
        pltpu.VMEM((EB, D), jnp.float32),
        pltpu.SemaphoreType.DMA,
        pltpu.SemaphoreType.DMA,
        pltpu.SemaphoreType.DMA,
        pltpu.SemaphoreType.DMA,
    ),
)
def _edge_sc(x_hbm, rel_hbm, keys_hbm, srcs_hbm, attr_hbm, noffs_hbm,
             sum_hbm, sq_hbm, mx_hbm, mn_hbm, deg_hbm,
             acc_s, acc_q, acc_mx, acc_mn, acc_dg, rel_v, noffs_v,
             key0, key1, att0, att1, idx0, idx1, rows0, rows1,
             semk0, semk1, semg0, semg1):
    keyb = (key0, key1)
    attb = (att0, att1)
    idxb = (idx0, idx1)
    rowsb = (rows0, rows1)
    semk = (semk0, semk1)
    semg = (semg0, semg1)
    wid = lax.axis_index("s") * 2 + lax.axis_index("c")
    wbase = pl.multiple_of(wid * NPW, 8)
    pltpu.sync_copy(rel_hbm, rel_v)
    pltpu.sync_copy(noffs_hbm.at[pl.ds(wbase, NPW + 16)], noffs_v)
    zero = jnp.zeros((16,), jnp.float32)
    ninf = jnp.full((16,), -jnp.inf, jnp.float32)
    pinf = jnp.full((16,), jnp.inf, jnp.float32)

    def fire_ks(k, buf, ast):
        sta = pl.multiple_of(ast + k * EB, 8)
        pltpu.async_copy(keys_hbm.at[pl.ds(sta, EB)],
                         keyb[buf].at[pl.ds(0, EB)], semk[buf])
        pltpu.async_copy(attr_hbm.at[pl.ds(sta, EB)],
                         attb[buf].at[pl.ds(0, EB)], semk[buf])
        pltpu.async_copy(srcs_hbm.at[pl.ds(sta, EB)], idxb[buf], semk[buf])

    def wait_ks(k, buf, ast):
        sta = pl.multiple_of(ast + k * EB, 8)
        pltpu.make_async_copy(keys_hbm.at[pl.ds(sta, EB)],
                              keyb[buf].at[pl.ds(0, EB)], semk[buf]).wait()
        pltpu.make_async_copy(attr_hbm.at[pl.ds(sta, EB)],
                              attb[buf].at[pl.ds(0, EB)], semk[buf]).wait()
        pltpu.make_async_copy(srcs_hbm.at[pl.ds(sta, EB)], idxb[buf],
                              semk[buf]).wait()

    def fire_g(buf):
        pltpu.async_copy(x_hbm.at[idxb[buf]], rowsb[buf], semg[buf])

    def wait_g(buf):
        pltpu.make_async_copy(x_hbm.at[idxb[buf]], rowsb[buf],
                              semg[buf]).wait()

    @pl.loop(wid * BPT, wid * BPT + BPT)
    def _bucket(b):
        base = pl.multiple_of(b * SZ, 8)
        lb = (b - wid * BPT) * SZ

        @plsc.parallel_loop(0, SZ)
        def _zrow(r):
            for j in range(8):
                sl = pl.ds(16 * j, 16)
                acc_s[r, sl] = zero
                acc_q[r, sl] = zero
                acc_mx[r, sl] = ninf
                acc_mn[r, sl] = pinf
            acc_dg[r, :] = zero

        def run_range(nof):
            start = nof[pl.ds(lb, 16)][0]
            stop = nof[pl.ds(lb + SZ, 16)][0]
            ast = (start // 8) * 8
            nb = (stop - ast + EB - 1) // EB

            @pl.when(nb > 0)
            def _prologue():
                fire_ks(0, 0, ast)
                wait_ks(0, 0, ast)
                fire_g(0)

                @pl.when(nb > 1)
                def _():
                    fire_ks(1, 1, ast)

            nbp = (nb + 1) // 2

            @pl.loop(0, nbp)
            def _pair(i):
                for half in range(2):
                    kk = i * 2 + half
                    buf = half

                    @pl.when(kk < nb)
                    def _do():
                        wait_g(buf)

                        @pl.when(kk + 1 < nb)
                        def _():
                            wait_ks(kk + 1, 1 - buf, ast)
                            fire_g(1 - buf)

                        st = ast + kk * EB
                        lo = jnp.maximum(start - st, 0)
                        hi = jnp.minimum(stop - st, EB)

                        @pl.when(hi > lo)
                        def _compute():
                            dlo = keyb[buf][pl.ds(lo, 16)][0]
                            dhi = keyb[buf][pl.ds(hi - 1, 16)][0]

                            @pl.loop(dlo, dhi + 1)
                            def _node(d):
                                dl = d - base
                                nv = nof[pl.ds(d - wbase, 16)]
                                elo = jnp.maximum(nv[0] - st, 0)
                                ehi = jnp.minimum(nv[1] - st, EB)
                                init = (tuple(zero for _ in range(8)),
                                        tuple(zero for _ in range(8)),
                                        tuple(ninf for _ in range(8)),
                                        tuple(pinf for _ in range(8)))

                                @plsc.parallel_loop(elo, ehi, unroll=2,
                                                    carry=init)
                                def _edge(e, c):
                                    cs, cq, cmx, cmn = c
                                    a = attb[buf][pl.ds(e, 16)][0]
                                    ns, nq, nmx, nmn = [], [], [], []
                                    for j in range(8):
                                        sl = pl.ds(16 * j, 16)
                                        m = rowsb[buf][e, sl] * rel_v[a, sl]
                                        ns.append(cs[j] + m)
                                        nq.append(cq[j] + m * m)
                                        nmx.append(jnp.maximum(cmx[j], m))
                                        nmn.append(jnp.minimum(cmn[j], m))
                                    return (tuple(ns), tuple(nq), tuple(nmx),
                                            tuple(nmn))

                                fs, fq, fmx, fmn = _edge
                                for j in range(8):
                                    sl = pl.ds(16 * j, 16)
                                    acc_s[dl, sl] = acc_s[dl, sl] + fs[j]
                                    acc_q[dl, sl] = acc_q[dl, sl] + fq[j]
                                    acc_mx[dl, sl] = jnp.maximum(
                                        acc_mx[dl, sl], fmx[j])
                                    acc_mn[dl, sl] = jnp.minimum(
                                        acc_mn[dl, sl], fmn[j])
                                cntf = (ehi - elo).astype(jnp.float32)
                                acc_dg[dl, :] = acc_dg[dl, :] + cntf

                        @pl.when(kk + 2 < nb)
                        def _refill():
                            fire_ks(kk + 2, buf, ast)

        run_range(noffs_v)

        pltpu.sync_copy(acc_s.at[pl.ds(0, SZ)], sum_hbm.at[pl.ds(base, SZ)])
        pltpu.sync_copy(acc_q.at[pl.ds(0, SZ)], sq_hbm.at[pl.ds(base, SZ)])
        pltpu.sync_copy(acc_mx.at[pl.ds(0, SZ)], mx_hbm.at[pl.ds(base, SZ)])
        pltpu.sync_copy(acc_mn.at[pl.ds(0, SZ)], mn_hbm.at[pl.ds(base, SZ)])
        pltpu.sync_copy(acc_dg.at[pl.ds(0, SZ)], deg_hbm.at[pl.ds(base, SZ)])


def _node_tc(x_ref, s_ref, q_ref, mx_ref, mn_ref, aux_ref, qry_ref, w_ref,
             b_ref, o_ref):
    aux = aux_ref[...]
    invd = aux[:, 0:1]
    s1 = aux[:, 1:2]
    s2 = aux[:, 2:3]
    cnt = aux[:, 3:4]
    qv = qry_ref[0:1, :]
    bnd = cnt * qv
    mean = (s_ref[...] + bnd) * invd
    sqm = (q_ref[...] + bnd * bnd) * invd
    mxv = jnp.maximum(mx_ref[...], bnd)
    mnv = jnp.minimum(mn_ref[...], bnd)
    std = jnp.sqrt(jnp.clip(sqm - mean * mean, 1e-6, None))
    acc = jnp.dot(x_ref[...], w_ref[0:D, :], preferred_element_type=jnp.float32)
    k = 1
    for stat in (mean, mxv, mnv, std):
        acc = acc + jnp.dot(stat, w_ref[k * D:(k + 1) * D, :],
                            preferred_element_type=jnp.float32)
        k += 1
        acc = acc + jnp.dot(stat * s1, w_ref[k * D:(k + 1) * D, :],
                            preferred_element_type=jnp.float32)
        k += 1
        acc = acc + jnp.dot(stat * s2, w_ref[k * D:(k + 1) * D, :],
                            preferred_element_type=jnp.float32)
        k += 1
    o_ref[...] = jnp.maximum(acc + b_ref[0:1, :], 0.0)


def _node_call(x, s, q, mx, mn, aux, q8, wre, b8):
    blk = 64
    grid = NPAD // blk

    def bs():
        return pl.BlockSpec((blk, D), lambda i: (i, 0))

    return pl.pallas_call(
        _node_tc,
        grid=(grid,),
        in_specs=[bs(), bs(), bs(), bs(), bs(), bs(),
                  pl.BlockSpec((8, D), lambda i: (0, 0)),
                  pl.BlockSpec((13 * D, D), lambda i: (0, 0)),
                  pl.BlockSpec((8, D), lambda i: (0, 0))],
        out_specs=bs(),
        out_shape=jax.ShapeDtypeStruct((NPAD, D), jnp.float32),
    )(x, s, q, mx, mn, aux, q8, wre, b8)


def _rel_tc(q_ref, w_ref, b_ref, o_ref):
    o_ref[...] = jnp.dot(q_ref[...], w_ref[...],
                         preferred_element_type=jnp.float32) + b_ref[...]


def _mlp_tc(t_ref, q_ref, wa_ref, wb_ref, bl_ref, w1_ref, b1_ref, w2_ref,
            b2_ref, o_ref):
    qrow = q_ref[0:1, :]
    h = (jnp.dot(t_ref[...], wa_ref[...], preferred_element_type=jnp.float32)
         + jnp.dot(qrow, wb_ref[...], preferred_element_type=jnp.float32)
         + bl_ref[0:1, :])
    h = jnp.maximum(h, 0.0)
    h = jnp.maximum(jnp.dot(h, w1_ref[...], preferred_element_type=jnp.float32)
                    + b1_ref[0:1, :], 0.0)
    o_ref[...] = jnp.dot(h, w2_ref[...],
                         preferred_element_type=jnp.float32) + b2_ref[0:1, :]


_PERM = np.concatenate(
    [np.arange(D)]
    + [D + 12 * np.arange(D) + 3 * c + j for c in range(4) for j in range(3)])


def kernel(h_index, r_index, t_index, hidden_states, rel_hidden_states,
           edge_index, edge_attr, score_text_embs, all_index, Wr0, br0, Wl0,
           bl0, Wr1, br1, Wl1, bl1, Wlin, blin, Wm1, bm1, Wm2, bm2):
    f32 = jnp.float32
    query = rel_hidden_states[r_index[0, 0]]
    q8 = jnp.broadcast_to(query[None, :], (8, D))

    wcat = jnp.concatenate([Wr0, Wr1], axis=1)
    bcat = jnp.broadcast_to(jnp.concatenate([br0, br1])[None, :], (8, 2 * NR2 * D))
    rel_out = pl.pallas_call(
        _rel_tc,
        out_shape=jax.ShapeDtypeStruct((8, 2 * NR2 * D), f32),
    )(q8, wcat, bcat)
    rel0 = rel_out[0, :NR2 * D].reshape(NR2, D)
    rel1 = rel_out[0, NR2 * D:].reshape(NR2, D)

    src = edge_index[0]
    dst = edge_index[1]
    keys = (dst.astype(jnp.uint32) * jnp.uint32(E)
            + src.astype(jnp.uint32) * jnp.uint32(32)
            + edge_attr.astype(jnp.uint32))
    sk = lax.sort(keys)
    rem = sk % jnp.uint32(E)
    sdst = (sk // jnp.uint32(E)).astype(jnp.int32)
    ssrc = (rem // jnp.uint32(32)).astype(jnp.int32)
    sattr = (rem % jnp.uint32(32)).astype(jnp.int32)
    nbounds = jnp.arange(NPAD + 1, dtype=jnp.uint32) * jnp.uint32(E)
    noffs = jnp.searchsorted(sk, nbounds).astype(jnp.int32)
    noffs_p = jnp.full((NPAD + 16,), E, jnp.int32).at[:NPAD + 1].set(noffs)
    skp = jnp.pad(sdst, (0, 2 * EB))
    ssp = jnp.pad(ssrc, (0, 2 * EB))
    sap = jnp.pad(sattr, (0, 2 * EB))

    cnt = jnp.zeros((NPAD, 1), f32).at[h_index[0]].add(1.0)
    x0 = jnp.pad(hidden_states + score_text_embs, ((0, NPAD - N), (0, 0)))

    wre0 = Wl0[_PERM]
    wre1 = Wl1[_PERM]
    b0_8 = jnp.broadcast_to(bl0[None, :], (8, D))
    b1_8 = jnp.broadcast_to(bl1[None, :], (8, D))

    s0, q0, mx0, mn0, dg0 = _edge_sc(x0, rel0, skp, ssp, sap, noffs_p)

    degf = dg0[:, 0:1] + 1.0
    sm = jnp.mean(jnp.log(degf[:N]))
    scale = jnp.log(degf) / (sm + 1e-10)
    aux4 = jnp.concatenate(
        [1.0 / degf, scale, 1.0 / jnp.clip(scale, 0.01, None), cnt], axis=1)
    aux = jnp.pad(aux4, ((0, 0), (0, D - 4)))

    x1 = _node_call(x0, s0, q0, mx0, mn0, aux, q8, wre0, b0_8)

    s1_, q1_, mx1, mn1, _ = _edge_sc(x1, rel1, skp, ssp, sap, noffs_p)
    x2 = _node_call(x1, s1_, q1_, mx1, mn1, aux, q8, wre1, b1_8)

    tails = jnp.pad(x2[t_index[0]], ((0, 40 - NEG), (0, 0)))
    w2p = jnp.pad(Wm2, ((0, 0), (0, D - 1)))
    b2p = jnp.broadcast_to(jnp.pad(bm2, (0, D - 1))[None, :], (8, D))
    out = pl.pallas_call(
        _mlp_tc,
        out_shape=jax.ShapeDtypeStruct((40, D), f32),
    )(tails, q8, Wlin[:D], Wlin[D:], jnp.broadcast_to(blin[None, :], (8, D)),
      Wm1, jnp.broadcast_to(bm1[None, :], (8, 2 * D)), w2p, b2p)
    return out[:NEG, 0].reshape(1, NEG)
